# baseline (speedup 1.0000x reference)
import jax
import jax.numpy as jnp
from jax.experimental import pallas as pl
from jax.experimental.pallas import tpu as pltpu

_B, _N, _H, _E, _MSG, _PASSES = 16, 64, 128, 16, 128, 3
_G = 8
_F32 = jnp.float32


def _mpnn_body(nodes_ref, e2_ref, W_edge_ref, W_node_ref, b_na_ref,
               W_zr_ref, U_zr_ref, b_zr_ref, Wh_ref, Uh_ref, bh_ref,
               W_gate_ref, b_gate_ref, W_er_ref, b_er_ref,
               W_out_ref, b_out_ref, out_ref):
    g, n = _G, _N
    nodes = nodes_ref[...].reshape(g * n, _H)
    e2 = e2_ref[...].reshape(g * n * n, _E)

    ep_emb = jnp.dot(e2, W_edge_ref[:, :_MSG], preferred_element_type=_F32
                     ).reshape(g, n, n, _MSG).astype(jnp.bfloat16)
    madj = jnp.dot(e2, W_edge_ref[:, 2 * _MSG:], preferred_element_type=_F32
                   ).reshape(g, n, n, _MSG)
    ep_att = jnp.dot(e2, W_edge_ref[:, _MSG:2 * _MSG],
                     preferred_element_type=_F32).reshape(g, n, n, _MSG)
    P = jnp.exp(jnp.where(madj != 0.0, ep_att, -1e30)).astype(jnp.bfloat16)
    node_mask = (jnp.max(madj, axis=2) != 0.0).reshape(g * n, _MSG)

    b_na = b_na_ref[0]
    b_zr = b_zr_ref[0]
    bh = bh_ref[0]
    hidden = nodes
    for _ in range(_PASSES):
        hb = hidden.astype(jnp.bfloat16)
        hn = jnp.dot(hb, W_node_ref[...],
                     preferred_element_type=_F32) + b_na
        hp = hn[:, :_MSG].astype(jnp.bfloat16).reshape(g, 1, n, _MSG)
        A = jnp.exp(hn[:, _MSG:]).astype(jnp.bfloat16).reshape(g, 1, n, _MSG)
        pm = A * P
        emb = jnp.maximum(hp + ep_emb, jnp.bfloat16(0))
        s = jnp.sum(pm, axis=2, dtype=_F32)
        msum = jnp.sum(pm * emb, axis=2, dtype=_F32)
        msgs = (msum / jnp.maximum(s, 1e-30)).reshape(g * n, _MSG)

        mb = msgs.astype(jnp.bfloat16)
        zr = jax.nn.sigmoid(
            jnp.dot(mb, W_zr_ref[...], preferred_element_type=_F32)
            + jnp.dot(hb, U_zr_ref[...], preferred_element_type=_F32)
            + b_zr)
        z = zr[:, :_H]
        r = zr[:, _H:]
        htil = jnp.tanh(
            jnp.dot(mb, Wh_ref[...], preferred_element_type=_F32)
            + jnp.dot((r * hidden).astype(jnp.bfloat16), Uh_ref[...],
                      preferred_element_type=_F32)
            + bh)
        hidden_new = (1.0 - z) * hidden + z * htil
        hidden = jnp.where(node_mask, hidden_new, hidden)

    gate = jax.nn.sigmoid(
        jnp.dot(hidden, W_gate_ref[:_H, :], preferred_element_type=_F32)
        + jnp.dot(nodes, W_gate_ref[_H:, :], preferred_element_type=_F32)
        + b_gate_ref[0])
    embr = jnp.dot(hidden, W_er_ref[...],
                   preferred_element_type=_F32) + b_er_ref[0]
    gp = jnp.where(node_mask, gate * embr, 0.0).reshape(g, n, _H)
    graph_emb = jnp.sum(gp, axis=1)
    out = jnp.dot(jnp.maximum(graph_emb, 0.0), W_out_ref[...],
                  preferred_element_type=_F32) + b_out_ref[0]
    out_ref[...] = out.reshape(g, 1, _H)


def kernel(nodes, edges, W_emb, b_emb, W_att, b_att, Wz, Uz, bz, Wr, Ur, br,
           Wh, Uh, bh, W_gate, b_gate, W_er, b_er, W_out, b_out):
    e2 = edges.reshape(_B, _N * _N, _E).astype(jnp.bfloat16)
    W_edge = jnp.concatenate(
        [W_emb[_H:], W_att[_H:], jnp.ones((_E, _MSG), _F32)],
        axis=1).astype(jnp.bfloat16)
    W_node = jnp.concatenate([W_emb[:_H], W_att[:_H]],
                             axis=1).astype(jnp.bfloat16)
    b_na = jnp.concatenate([b_emb, b_att]).reshape(1, 2 * _MSG)
    W_zr = jnp.concatenate([Wz, Wr], axis=1).astype(jnp.bfloat16)
    U_zr = jnp.concatenate([Uz, Ur], axis=1).astype(jnp.bfloat16)
    Wh = Wh.astype(jnp.bfloat16)
    Uh = Uh.astype(jnp.bfloat16)
    b_zr = jnp.concatenate([bz, br]).reshape(1, 2 * _H)
    bh2 = bh.reshape(1, _H)
    b_gate2 = b_gate.reshape(1, _H)
    b_er2 = b_er.reshape(1, _H)
    b_out2 = b_out.reshape(1, _H)

    def full(shape):
        return pl.BlockSpec(shape, lambda b: (0,) * len(shape))

    grid_spec = pl.GridSpec(
        grid=(_B // _G,),
        in_specs=[
            pl.BlockSpec((_G, _N, _H), lambda b: (b, 0, 0)),
            pl.BlockSpec((_G, _N * _N, _E), lambda b: (b, 0, 0)),
            full(W_edge.shape), full(W_node.shape), full(b_na.shape),
            full(W_zr.shape), full(U_zr.shape), full(b_zr.shape),
            full(Wh.shape), full(Uh.shape), full(bh2.shape),
            full(W_gate.shape), full(b_gate2.shape),
            full(W_er.shape), full(b_er2.shape),
            full(W_out.shape), full(b_out2.shape),
        ],
        out_specs=pl.BlockSpec((_G, 1, _H), lambda b: (b, 0, 0)),
    )

    out = pl.pallas_call(
        _mpnn_body,
        grid_spec=grid_spec,
        out_shape=jax.ShapeDtypeStruct((_B, 1, _H), _F32),
        compiler_params=pltpu.CompilerParams(
            dimension_semantics=("arbitrary",)),
    )(nodes, e2, W_edge, W_node, b_na, W_zr, U_zr, b_zr,
      Wh, Uh, bh2, W_gate, b_gate2, W_er, b_er2, W_out, b_out2)
    return out.reshape(_B, _H)


# speedup vs baseline: 1.0843x; 1.0843x over previous
import jax
import jax.numpy as jnp
from jax.experimental import pallas as pl
from jax.experimental.pallas import tpu as pltpu

_B, _N, _H, _E, _MSG, _PASSES = 16, 64, 128, 16, 128, 3
_G = 8
_F32 = jnp.float32


def _mpnn_body(nodes_ref, e2_ref, W_edge_ref, W_node_ref, b_na_ref,
               W_zr_ref, U_zr_ref, b_zr_ref, Wh_ref, Uh_ref, bh_ref,
               W_gate_ref, b_gate_ref, W_er_ref, b_er_ref,
               W_out_ref, b_out_ref, out_ref):
    g, n = _G, _N
    nodes = nodes_ref[...].reshape(g * n, _H)
    e2 = e2_ref[...].reshape(g * n * n, _E)

    ep_emb = jnp.dot(e2, W_edge_ref[:, :_MSG], preferred_element_type=_F32
                     ).reshape(g, n, n, _MSG).astype(jnp.bfloat16)
    madj = jnp.dot(e2, W_edge_ref[:, 2 * _MSG:], preferred_element_type=_F32
                   ).reshape(g, n, n, _MSG)
    ep_att = jnp.dot(e2, W_edge_ref[:, _MSG:2 * _MSG],
                     preferred_element_type=_F32).reshape(g, n, n, _MSG)
    P = jnp.exp(jnp.where(madj != 0.0, ep_att, -1e30)).astype(jnp.bfloat16)
    node_mask = (jnp.max(madj, axis=2) != 0.0).reshape(g * n, _MSG)

    b_na = b_na_ref[0]
    b_zr = b_zr_ref[0]
    bh = bh_ref[0]
    hidden = nodes
    ones_j = jnp.ones((g * n, 1, n), jnp.bfloat16)
    dn = (((2,), (1,)), ((0,), (0,)))
    for _ in range(_PASSES):
        hb = hidden.astype(jnp.bfloat16)
        hn = jnp.dot(hb, W_node_ref[...],
                     preferred_element_type=_F32) + b_na
        hp = hn[:, :_MSG].astype(jnp.bfloat16).reshape(g, 1, n, _MSG)
        A = jnp.exp(hn[:, _MSG:]).astype(jnp.bfloat16).reshape(g, 1, n, _MSG)
        pm = A * P
        emb = jnp.maximum(hp + ep_emb, jnp.bfloat16(0))
        pm3 = pm.reshape(g * n, n, _MSG)
        s = jax.lax.dot_general(ones_j, pm3, dn,
                                preferred_element_type=_F32)
        msum = jax.lax.dot_general(ones_j, (pm * emb).reshape(g * n, n, _MSG),
                                   dn, preferred_element_type=_F32)
        msgs = (msum / jnp.maximum(s, 1e-30)).reshape(g * n, _MSG)

        mb = msgs.astype(jnp.bfloat16)
        zr = jax.nn.sigmoid(
            jnp.dot(mb, W_zr_ref[...], preferred_element_type=_F32)
            + jnp.dot(hb, U_zr_ref[...], preferred_element_type=_F32)
            + b_zr)
        z = zr[:, :_H]
        r = zr[:, _H:]
        htil = jnp.tanh(
            jnp.dot(mb, Wh_ref[...], preferred_element_type=_F32)
            + jnp.dot((r * hidden).astype(jnp.bfloat16), Uh_ref[...],
                      preferred_element_type=_F32)
            + bh)
        hidden_new = (1.0 - z) * hidden + z * htil
        hidden = jnp.where(node_mask, hidden_new, hidden)

    gate = jax.nn.sigmoid(
        jnp.dot(hidden, W_gate_ref[:_H, :], preferred_element_type=_F32)
        + jnp.dot(nodes, W_gate_ref[_H:, :], preferred_element_type=_F32)
        + b_gate_ref[0])
    embr = jnp.dot(hidden, W_er_ref[...],
                   preferred_element_type=_F32) + b_er_ref[0]
    gp = jnp.where(node_mask, gate * embr, 0.0).reshape(g, n, _H)
    graph_emb = jnp.sum(gp, axis=1)
    out = jnp.dot(jnp.maximum(graph_emb, 0.0), W_out_ref[...],
                  preferred_element_type=_F32) + b_out_ref[0]
    out_ref[...] = out.reshape(g, 1, _H)


def kernel(nodes, edges, W_emb, b_emb, W_att, b_att, Wz, Uz, bz, Wr, Ur, br,
           Wh, Uh, bh, W_gate, b_gate, W_er, b_er, W_out, b_out):
    e2 = edges.reshape(_B, _N * _N, _E).astype(jnp.bfloat16)
    W_edge = jnp.concatenate(
        [W_emb[_H:], W_att[_H:], jnp.ones((_E, _MSG), _F32)],
        axis=1).astype(jnp.bfloat16)
    W_node = jnp.concatenate([W_emb[:_H], W_att[:_H]],
                             axis=1).astype(jnp.bfloat16)
    b_na = jnp.concatenate([b_emb, b_att]).reshape(1, 2 * _MSG)
    W_zr = jnp.concatenate([Wz, Wr], axis=1).astype(jnp.bfloat16)
    U_zr = jnp.concatenate([Uz, Ur], axis=1).astype(jnp.bfloat16)
    Wh = Wh.astype(jnp.bfloat16)
    Uh = Uh.astype(jnp.bfloat16)
    b_zr = jnp.concatenate([bz, br]).reshape(1, 2 * _H)
    bh2 = bh.reshape(1, _H)
    b_gate2 = b_gate.reshape(1, _H)
    b_er2 = b_er.reshape(1, _H)
    b_out2 = b_out.reshape(1, _H)

    def full(shape):
        return pl.BlockSpec(shape, lambda b: (0,) * len(shape))

    grid_spec = pl.GridSpec(
        grid=(_B // _G,),
        in_specs=[
            pl.BlockSpec((_G, _N, _H), lambda b: (b, 0, 0)),
            pl.BlockSpec((_G, _N * _N, _E), lambda b: (b, 0, 0)),
            full(W_edge.shape), full(W_node.shape), full(b_na.shape),
            full(W_zr.shape), full(U_zr.shape), full(b_zr.shape),
            full(Wh.shape), full(Uh.shape), full(bh2.shape),
            full(W_gate.shape), full(b_gate2.shape),
            full(W_er.shape), full(b_er2.shape),
            full(W_out.shape), full(b_out2.shape),
        ],
        out_specs=pl.BlockSpec((_G, 1, _H), lambda b: (b, 0, 0)),
    )

    out = pl.pallas_call(
        _mpnn_body,
        grid_spec=grid_spec,
        out_shape=jax.ShapeDtypeStruct((_B, 1, _H), _F32),
        compiler_params=pltpu.CompilerParams(
            dimension_semantics=("arbitrary",)),
    )(nodes, e2, W_edge, W_node, b_na, W_zr, U_zr, b_zr,
      Wh, Uh, bh2, W_gate, b_gate2, W_er, b_er2, W_out, b_out2)
    return out.reshape(_B, _H)


# speedup vs baseline: 1.3127x; 1.2107x over previous
import jax
import jax.numpy as jnp
from jax.experimental import pallas as pl
from jax.experimental.pallas import tpu as pltpu

_B, _N, _H, _E, _MSG, _PASSES = 16, 64, 128, 16, 128, 3
_G = 8
_F32 = jnp.float32


def _mpnn_body(nodes_ref, e2_ref, W_edge_ref, W_node_ref, b_na_ref,
               W_zr_ref, U_zr_ref, b_zr_ref, Wh_ref, Uh_ref, bh_ref,
               W_gate_ref, b_gate_ref, W_er_ref, b_er_ref,
               W_out_ref, b_out_ref, out_ref):
    g, n = _G, _N
    nodes = nodes_ref[...].reshape(g * n, _H)
    e2 = e2_ref[...].reshape(g * n * n, _E)

    ep_emb = jnp.dot(e2, W_edge_ref[:, :_MSG], preferred_element_type=_F32
                     ).reshape(g, n, n, _MSG).astype(jnp.bfloat16)
    madj = jnp.dot(e2, W_edge_ref[:, 2 * _MSG:], preferred_element_type=_F32
                   ).reshape(g, n, n, _MSG)
    ep_att = jnp.dot(e2, W_edge_ref[:, _MSG:2 * _MSG],
                     preferred_element_type=_F32).reshape(g, n, n, _MSG)
    P = jnp.exp(jnp.where(madj != 0.0, ep_att, -1e30)).astype(jnp.bfloat16)
    node_mask = (jnp.max(madj, axis=2) != 0.0).reshape(g * n, _MSG)

    b_na = b_na_ref[0]
    b_zr = b_zr_ref[0]
    bh = bh_ref[0]
    hidden = nodes
    ones_j = jnp.ones((g * n, 1, n), jnp.bfloat16)
    dn = (((2,), (1,)), ((0,), (0,)))
    for _ in range(_PASSES):
        hb = hidden.astype(jnp.bfloat16)
        hn = jnp.dot(hb, W_node_ref[...],
                     preferred_element_type=_F32) + b_na
        hp = hn[:, :_MSG].astype(jnp.bfloat16).reshape(g, 1, n, _MSG)
        A = jnp.exp(hn[:, _MSG:]).astype(jnp.bfloat16).reshape(g, 1, n, _MSG)
        pm = A * P
        emb = jnp.maximum(hp + ep_emb, jnp.bfloat16(0))
        rhs = jnp.concatenate(
            [(pm * emb).reshape(g * n, n, _MSG),
             pm.reshape(g * n, n, _MSG)], axis=2)
        both = jax.lax.dot_general(ones_j, rhs, dn,
                                   preferred_element_type=_F32)
        msum = both[:, 0, :_MSG]
        s = both[:, 0, _MSG:]
        msgs = msum / jnp.maximum(s, 1e-30)

        mb = msgs.astype(jnp.bfloat16)
        zr = jax.nn.sigmoid(
            jnp.dot(mb, W_zr_ref[...], preferred_element_type=_F32)
            + jnp.dot(hb, U_zr_ref[...], preferred_element_type=_F32)
            + b_zr)
        z = zr[:, :_H]
        r = zr[:, _H:]
        htil = jnp.tanh(
            jnp.dot(mb, Wh_ref[...], preferred_element_type=_F32)
            + jnp.dot((r * hidden).astype(jnp.bfloat16), Uh_ref[...],
                      preferred_element_type=_F32)
            + bh)
        hidden_new = (1.0 - z) * hidden + z * htil
        hidden = jnp.where(node_mask, hidden_new, hidden)

    gate = jax.nn.sigmoid(
        jnp.dot(hidden, W_gate_ref[:_H, :], preferred_element_type=_F32)
        + jnp.dot(nodes, W_gate_ref[_H:, :], preferred_element_type=_F32)
        + b_gate_ref[0])
    embr = jnp.dot(hidden, W_er_ref[...],
                   preferred_element_type=_F32) + b_er_ref[0]
    gp = jnp.where(node_mask, gate * embr, 0.0).reshape(g, n, _H)
    graph_emb = jnp.sum(gp, axis=1)
    out = jnp.dot(jnp.maximum(graph_emb, 0.0), W_out_ref[...],
                  preferred_element_type=_F32) + b_out_ref[0]
    out_ref[...] = out.reshape(g, 1, _H)


def kernel(nodes, edges, W_emb, b_emb, W_att, b_att, Wz, Uz, bz, Wr, Ur, br,
           Wh, Uh, bh, W_gate, b_gate, W_er, b_er, W_out, b_out):
    e2 = edges.reshape(_B, _N * _N, _E).astype(jnp.bfloat16)
    W_edge = jnp.concatenate(
        [W_emb[_H:], W_att[_H:], jnp.ones((_E, _MSG), _F32)],
        axis=1).astype(jnp.bfloat16)
    W_node = jnp.concatenate([W_emb[:_H], W_att[:_H]],
                             axis=1).astype(jnp.bfloat16)
    b_na = jnp.concatenate([b_emb, b_att]).reshape(1, 2 * _MSG)
    W_zr = jnp.concatenate([Wz, Wr], axis=1).astype(jnp.bfloat16)
    U_zr = jnp.concatenate([Uz, Ur], axis=1).astype(jnp.bfloat16)
    Wh = Wh.astype(jnp.bfloat16)
    Uh = Uh.astype(jnp.bfloat16)
    b_zr = jnp.concatenate([bz, br]).reshape(1, 2 * _H)
    bh2 = bh.reshape(1, _H)
    b_gate2 = b_gate.reshape(1, _H)
    b_er2 = b_er.reshape(1, _H)
    b_out2 = b_out.reshape(1, _H)

    def full(shape):
        return pl.BlockSpec(shape, lambda b: (0,) * len(shape))

    grid_spec = pl.GridSpec(
        grid=(_B // _G,),
        in_specs=[
            pl.BlockSpec((_G, _N, _H), lambda b: (b, 0, 0)),
            pl.BlockSpec((_G, _N * _N, _E), lambda b: (b, 0, 0)),
            full(W_edge.shape), full(W_node.shape), full(b_na.shape),
            full(W_zr.shape), full(U_zr.shape), full(b_zr.shape),
            full(Wh.shape), full(Uh.shape), full(bh2.shape),
            full(W_gate.shape), full(b_gate2.shape),
            full(W_er.shape), full(b_er2.shape),
            full(W_out.shape), full(b_out2.shape),
        ],
        out_specs=pl.BlockSpec((_G, 1, _H), lambda b: (b, 0, 0)),
    )

    out = pl.pallas_call(
        _mpnn_body,
        grid_spec=grid_spec,
        out_shape=jax.ShapeDtypeStruct((_B, 1, _H), _F32),
        compiler_params=pltpu.CompilerParams(
            dimension_semantics=("arbitrary",)),
    )(nodes, e2, W_edge, W_node, b_na, W_zr, U_zr, b_zr,
      Wh, Uh, bh2, W_gate, b_gate2, W_er, b_er2, W_out, b_out2)
    return out.reshape(_B, _H)


# speedup vs baseline: 1.3129x; 1.0001x over previous
import jax
import jax.numpy as jnp
from jax.experimental import pallas as pl
from jax.experimental.pallas import tpu as pltpu

_B, _N, _H, _E, _MSG, _PASSES = 16, 64, 128, 16, 128, 3
_G = 8
_F32 = jnp.float32


def _mpnn_body(nodes_ref, e2_ref, W_edge_ref, W_node_ref, b_na_ref,
               W_zr_ref, U_zr_ref, b_zr_ref, Wh_ref, Uh_ref, bh_ref,
               W_gate_ref, b_gate_ref, W_er_ref, b_er_ref,
               W_out_ref, b_out_ref, out_ref):
    g, n = _G, _N
    nodes = nodes_ref[...].reshape(g * n, _H)
    e2 = e2_ref[...].reshape(g * n * n, _E)

    ep_emb = jnp.dot(e2, W_edge_ref[:, :_MSG], preferred_element_type=_F32
                     ).reshape(g, n, n, _MSG).astype(jnp.bfloat16)
    madj = jnp.dot(e2, W_edge_ref[:, 2 * _MSG:], preferred_element_type=_F32
                   ).reshape(g, n, n, _MSG)
    ep_att = jnp.dot(e2, W_edge_ref[:, _MSG:2 * _MSG],
                     preferred_element_type=_F32).reshape(g, n, n, _MSG)
    P = jnp.exp(jnp.where(madj != 0.0, ep_att, -1e30)).astype(jnp.bfloat16)

    b_na = b_na_ref[0]
    b_zr = b_zr_ref[0]
    bh = bh_ref[0]
    hidden = nodes
    ones_j = jnp.ones((g * n, 1, n), jnp.bfloat16)
    dn = (((2,), (1,)), ((0,), (0,)))
    for _ in range(_PASSES):
        hb = hidden.astype(jnp.bfloat16)
        hn = jnp.dot(hb, W_node_ref[...],
                     preferred_element_type=_F32) + b_na
        hp = hn[:, :_MSG].astype(jnp.bfloat16).reshape(g, 1, n, _MSG)
        A = jnp.exp(hn[:, _MSG:]).astype(jnp.bfloat16).reshape(g, 1, n, _MSG)
        pm = A * P
        emb = jnp.maximum(hp + ep_emb, jnp.bfloat16(0))
        rhs = jnp.concatenate(
            [(pm * emb).reshape(g * n, n, _MSG),
             pm.reshape(g * n, n, _MSG)], axis=2)
        both = jax.lax.dot_general(ones_j, rhs, dn,
                                   preferred_element_type=_F32)
        msum = both[:, 0, :_MSG]
        s = both[:, 0, _MSG:]
        node_mask = s != 0.0
        msgs = msum / jnp.maximum(s, 1e-30)

        mb = msgs.astype(jnp.bfloat16)
        zr = jax.nn.sigmoid(
            jnp.dot(mb, W_zr_ref[...], preferred_element_type=_F32)
            + jnp.dot(hb, U_zr_ref[...], preferred_element_type=_F32)
            + b_zr)
        z = zr[:, :_H]
        r = zr[:, _H:]
        htil = jnp.tanh(
            jnp.dot(mb, Wh_ref[...], preferred_element_type=_F32)
            + jnp.dot((r * hidden).astype(jnp.bfloat16), Uh_ref[...],
                      preferred_element_type=_F32)
            + bh)
        hidden_new = (1.0 - z) * hidden + z * htil
        hidden = jnp.where(node_mask, hidden_new, hidden)

    gate = jax.nn.sigmoid(
        jnp.dot(hidden, W_gate_ref[:_H, :], preferred_element_type=_F32)
        + jnp.dot(nodes, W_gate_ref[_H:, :], preferred_element_type=_F32)
        + b_gate_ref[0])
    embr = jnp.dot(hidden, W_er_ref[...],
                   preferred_element_type=_F32) + b_er_ref[0]
    gp = jnp.where(node_mask, gate * embr, 0.0).reshape(g, n, _H)
    graph_emb = jnp.sum(gp, axis=1)
    out = jnp.dot(jnp.maximum(graph_emb, 0.0), W_out_ref[...],
                  preferred_element_type=_F32) + b_out_ref[0]
    out_ref[...] = out.reshape(g, 1, _H)


def kernel(nodes, edges, W_emb, b_emb, W_att, b_att, Wz, Uz, bz, Wr, Ur, br,
           Wh, Uh, bh, W_gate, b_gate, W_er, b_er, W_out, b_out):
    e2 = edges.reshape(_B, _N * _N, _E).astype(jnp.bfloat16)
    W_edge = jnp.concatenate(
        [W_emb[_H:], W_att[_H:], jnp.ones((_E, _MSG), _F32)],
        axis=1).astype(jnp.bfloat16)
    W_node = jnp.concatenate([W_emb[:_H], W_att[:_H]],
                             axis=1).astype(jnp.bfloat16)
    b_na = jnp.concatenate([b_emb, b_att]).reshape(1, 2 * _MSG)
    W_zr = jnp.concatenate([Wz, Wr], axis=1).astype(jnp.bfloat16)
    U_zr = jnp.concatenate([Uz, Ur], axis=1).astype(jnp.bfloat16)
    Wh = Wh.astype(jnp.bfloat16)
    Uh = Uh.astype(jnp.bfloat16)
    b_zr = jnp.concatenate([bz, br]).reshape(1, 2 * _H)
    bh2 = bh.reshape(1, _H)
    b_gate2 = b_gate.reshape(1, _H)
    b_er2 = b_er.reshape(1, _H)
    b_out2 = b_out.reshape(1, _H)

    def full(shape):
        return pl.BlockSpec(shape, lambda b: (0,) * len(shape))

    grid_spec = pl.GridSpec(
        grid=(_B // _G,),
        in_specs=[
            pl.BlockSpec((_G, _N, _H), lambda b: (b, 0, 0)),
            pl.BlockSpec((_G, _N * _N, _E), lambda b: (b, 0, 0)),
            full(W_edge.shape), full(W_node.shape), full(b_na.shape),
            full(W_zr.shape), full(U_zr.shape), full(b_zr.shape),
            full(Wh.shape), full(Uh.shape), full(bh2.shape),
            full(W_gate.shape), full(b_gate2.shape),
            full(W_er.shape), full(b_er2.shape),
            full(W_out.shape), full(b_out2.shape),
        ],
        out_specs=pl.BlockSpec((_G, 1, _H), lambda b: (b, 0, 0)),
    )

    out = pl.pallas_call(
        _mpnn_body,
        grid_spec=grid_spec,
        out_shape=jax.ShapeDtypeStruct((_B, 1, _H), _F32),
        compiler_params=pltpu.CompilerParams(
            dimension_semantics=("arbitrary",)),
    )(nodes, e2, W_edge, W_node, b_na, W_zr, U_zr, b_zr,
      Wh, Uh, bh2, W_gate, b_gate2, W_er, b_er2, W_out, b_out2)
    return out.reshape(_B, _H)


# speedup vs baseline: 1.3891x; 1.0581x over previous
import jax
import jax.numpy as jnp
from jax.experimental import pallas as pl
from jax.experimental.pallas import tpu as pltpu

_B, _N, _H, _E, _MSG, _PASSES = 16, 64, 128, 16, 128, 3
_G = 8
_F32 = jnp.float32


def _mpnn_body(nodes_ref, e2_ref, W_edge_ref, W_node_ref, b_na_ref,
               W_zr_ref, U_zr_ref, b_zr_ref, Wh_ref, Uh_ref, bh_ref,
               W_gate_ref, b_gate_ref, W_er_ref, b_er_ref,
               W_out_ref, b_out_ref, out_ref):
    g, n = _G, _N
    nodes = nodes_ref[...].reshape(g * n, _H)
    e2 = e2_ref[...].reshape(g * n * n, _E)

    ep_emb = jnp.dot(e2, W_edge_ref[:, :_MSG], preferred_element_type=_F32
                     ).reshape(g, n, n, _MSG).astype(jnp.bfloat16)
    madj = jnp.dot(e2, W_edge_ref[:, 2 * _MSG:], preferred_element_type=_F32
                   ).reshape(g, n, n, _MSG)
    ep_att = jnp.dot(e2, W_edge_ref[:, _MSG:2 * _MSG],
                     preferred_element_type=_F32).reshape(g, n, n, _MSG)
    P = jnp.exp(jnp.where(madj != 0.0, ep_att, -1e30)).astype(jnp.bfloat16)

    b_na = b_na_ref[0]
    b_zr = b_zr_ref[0]
    bh = bh_ref[0]
    hidden = nodes
    ki = jax.lax.broadcasted_iota(jnp.int32, (8, 8 * n), 1) // n
    ri = jax.lax.broadcasted_iota(jnp.int32, (8, 8 * n), 0)
    sel8 = (ki == ri).astype(jnp.bfloat16)
    lhs8 = jnp.broadcast_to(sel8[None], (g * n // 8, 8, 8 * n))
    dn = (((2,), (1,)), ((0,), (0,)))
    for _ in range(_PASSES):
        hb = hidden.astype(jnp.bfloat16)
        hn = jnp.dot(hb, W_node_ref[...],
                     preferred_element_type=_F32) + b_na
        hp = hn[:, :_MSG].astype(jnp.bfloat16).reshape(g, 1, n, _MSG)
        A = jnp.exp(hn[:, _MSG:]).astype(jnp.bfloat16).reshape(g, 1, n, _MSG)
        pm = A * P
        emb = jnp.maximum(hp + ep_emb, jnp.bfloat16(0))
        rhs = jnp.concatenate(
            [(pm * emb).reshape(g * n // 8, 8 * n, _MSG),
             pm.reshape(g * n // 8, 8 * n, _MSG)], axis=2)
        both = jax.lax.dot_general(lhs8, rhs, dn,
                                   preferred_element_type=_F32)
        both2 = both.reshape(g * n, 2 * _MSG)
        msum = both2[:, :_MSG]
        s = both2[:, _MSG:]
        node_mask = s != 0.0
        msgs = msum / jnp.maximum(s, 1e-30)

        mb = msgs.astype(jnp.bfloat16)
        zr = jax.nn.sigmoid(
            jnp.dot(mb, W_zr_ref[...], preferred_element_type=_F32)
            + jnp.dot(hb, U_zr_ref[...], preferred_element_type=_F32)
            + b_zr)
        z = zr[:, :_H]
        r = zr[:, _H:]
        htil = jnp.tanh(
            jnp.dot(mb, Wh_ref[...], preferred_element_type=_F32)
            + jnp.dot((r * hidden).astype(jnp.bfloat16), Uh_ref[...],
                      preferred_element_type=_F32)
            + bh)
        hidden_new = (1.0 - z) * hidden + z * htil
        hidden = jnp.where(node_mask, hidden_new, hidden)

    gate = jax.nn.sigmoid(
        jnp.dot(hidden, W_gate_ref[:_H, :], preferred_element_type=_F32)
        + jnp.dot(nodes, W_gate_ref[_H:, :], preferred_element_type=_F32)
        + b_gate_ref[0])
    embr = jnp.dot(hidden, W_er_ref[...],
                   preferred_element_type=_F32) + b_er_ref[0]
    gp = jnp.where(node_mask, gate * embr, 0.0).reshape(g, n, _H)
    graph_emb = jnp.sum(gp, axis=1)
    out = jnp.dot(jnp.maximum(graph_emb, 0.0), W_out_ref[...],
                  preferred_element_type=_F32) + b_out_ref[0]
    out_ref[...] = out.reshape(g, 1, _H)


def kernel(nodes, edges, W_emb, b_emb, W_att, b_att, Wz, Uz, bz, Wr, Ur, br,
           Wh, Uh, bh, W_gate, b_gate, W_er, b_er, W_out, b_out):
    e2 = edges.reshape(_B, _N * _N, _E).astype(jnp.bfloat16)
    W_edge = jnp.concatenate(
        [W_emb[_H:], W_att[_H:], jnp.ones((_E, _MSG), _F32)],
        axis=1).astype(jnp.bfloat16)
    W_node = jnp.concatenate([W_emb[:_H], W_att[:_H]],
                             axis=1).astype(jnp.bfloat16)
    b_na = jnp.concatenate([b_emb, b_att]).reshape(1, 2 * _MSG)
    W_zr = jnp.concatenate([Wz, Wr], axis=1).astype(jnp.bfloat16)
    U_zr = jnp.concatenate([Uz, Ur], axis=1).astype(jnp.bfloat16)
    Wh = Wh.astype(jnp.bfloat16)
    Uh = Uh.astype(jnp.bfloat16)
    b_zr = jnp.concatenate([bz, br]).reshape(1, 2 * _H)
    bh2 = bh.reshape(1, _H)
    b_gate2 = b_gate.reshape(1, _H)
    b_er2 = b_er.reshape(1, _H)
    b_out2 = b_out.reshape(1, _H)

    def full(shape):
        return pl.BlockSpec(shape, lambda b: (0,) * len(shape))

    grid_spec = pl.GridSpec(
        grid=(_B // _G,),
        in_specs=[
            pl.BlockSpec((_G, _N, _H), lambda b: (b, 0, 0)),
            pl.BlockSpec((_G, _N * _N, _E), lambda b: (b, 0, 0)),
            full(W_edge.shape), full(W_node.shape), full(b_na.shape),
            full(W_zr.shape), full(U_zr.shape), full(b_zr.shape),
            full(Wh.shape), full(Uh.shape), full(bh2.shape),
            full(W_gate.shape), full(b_gate2.shape),
            full(W_er.shape), full(b_er2.shape),
            full(W_out.shape), full(b_out2.shape),
        ],
        out_specs=pl.BlockSpec((_G, 1, _H), lambda b: (b, 0, 0)),
    )

    out = pl.pallas_call(
        _mpnn_body,
        grid_spec=grid_spec,
        out_shape=jax.ShapeDtypeStruct((_B, 1, _H), _F32),
        compiler_params=pltpu.CompilerParams(
            dimension_semantics=("arbitrary",)),
    )(nodes, e2, W_edge, W_node, b_na, W_zr, U_zr, b_zr,
      Wh, Uh, bh2, W_gate, b_gate2, W_er, b_er2, W_out, b_out2)
    return out.reshape(_B, _H)


# speedup vs baseline: 1.4241x; 1.0252x over previous
import jax
import jax.numpy as jnp
from jax.experimental import pallas as pl
from jax.experimental.pallas import tpu as pltpu

_B, _N, _H, _E, _MSG, _PASSES = 16, 64, 128, 16, 128, 3
_G = 8
_F32 = jnp.float32


def _mpnn_body(nodes_ref, e2_ref, W_edge_ref, W_node_ref, b_na_ref,
               W_zr_ref, b_zr_ref, Uh_ref,
               W_gate_ref, b_gate_ref, W_er_ref, b_er_ref,
               W_out_ref, b_out_ref, out_ref):
    g, n = _G, _N
    nodes = nodes_ref[...].reshape(g * n, _H)
    e2 = e2_ref[...].reshape(g * n * n, _E)

    ep_emb = jnp.dot(e2, W_edge_ref[:, :_MSG], preferred_element_type=_F32
                     ).reshape(g, n, n, _MSG).astype(jnp.bfloat16)
    madj = jnp.dot(e2, W_edge_ref[:, 2 * _MSG:], preferred_element_type=_F32
                   ).reshape(g, n, n, _MSG)
    ep_att = jnp.dot(e2, W_edge_ref[:, _MSG:2 * _MSG],
                     preferred_element_type=_F32).reshape(g, n, n, _MSG)
    P = jnp.exp(jnp.where(madj != 0.0, ep_att, -1e30)).astype(jnp.bfloat16)

    b_na = b_na_ref[0]
    b_zr = b_zr_ref[0]
    hidden = nodes
    ki = jax.lax.broadcasted_iota(jnp.int32, (8, 8 * n), 1) // n
    ri = jax.lax.broadcasted_iota(jnp.int32, (8, 8 * n), 0)
    sel8 = (ki == ri).astype(jnp.bfloat16)
    lhs8 = jnp.broadcast_to(sel8[None], (g * n // 8, 8, 8 * n))
    dn = (((2,), (1,)), ((0,), (0,)))
    for _ in range(_PASSES):
        hb = hidden.astype(jnp.bfloat16)
        hn = jnp.dot(hb, W_node_ref[...],
                     preferred_element_type=_F32) + b_na
        hp = hn[:, :_MSG].astype(jnp.bfloat16).reshape(g, 1, n, _MSG)
        A = jnp.exp(hn[:, _MSG:]).astype(jnp.bfloat16).reshape(g, 1, n, _MSG)
        pm = A * P
        emb = jnp.maximum(hp + ep_emb, jnp.bfloat16(0))
        rhs = jnp.concatenate(
            [(pm * emb).reshape(g * n // 8, 8 * n, _MSG),
             pm.reshape(g * n // 8, 8 * n, _MSG)], axis=2)
        both = jax.lax.dot_general(lhs8, rhs, dn,
                                   preferred_element_type=_F32)
        both2 = both.reshape(g * n, 2 * _MSG)
        msum = both2[:, :_MSG]
        s = both2[:, _MSG:]
        node_mask = s != 0.0
        msgs = msum / jnp.maximum(s, 1e-30)

        mb = msgs.astype(jnp.bfloat16)
        big = jnp.dot(jnp.concatenate([mb, hb], axis=1), W_zr_ref[...],
                      preferred_element_type=_F32) + b_zr
        zr = jax.nn.sigmoid(big[:, :2 * _H])
        z = zr[:, :_H]
        r = zr[:, _H:]
        htil = jnp.tanh(
            big[:, 2 * _H:]
            + jnp.dot((r * hidden).astype(jnp.bfloat16), Uh_ref[...],
                      preferred_element_type=_F32))
        hidden_new = (1.0 - z) * hidden + z * htil
        hidden = jnp.where(node_mask, hidden_new, hidden)

    gate = jax.nn.sigmoid(
        jnp.dot(hidden, W_gate_ref[:_H, :], preferred_element_type=_F32)
        + jnp.dot(nodes, W_gate_ref[_H:, :], preferred_element_type=_F32)
        + b_gate_ref[0])
    embr = jnp.dot(hidden, W_er_ref[...],
                   preferred_element_type=_F32) + b_er_ref[0]
    gp = jnp.where(node_mask, gate * embr, 0.0).reshape(g, n, _H)
    graph_emb = jnp.sum(gp, axis=1)
    out = jnp.dot(jnp.maximum(graph_emb, 0.0), W_out_ref[...],
                  preferred_element_type=_F32) + b_out_ref[0]
    out_ref[...] = out.reshape(g, 1, _H)


def kernel(nodes, edges, W_emb, b_emb, W_att, b_att, Wz, Uz, bz, Wr, Ur, br,
           Wh, Uh, bh, W_gate, b_gate, W_er, b_er, W_out, b_out):
    e2 = edges.reshape(_B, _N * _N, _E).astype(jnp.bfloat16)
    W_edge = jnp.concatenate(
        [W_emb[_H:], W_att[_H:], jnp.ones((_E, _MSG), _F32)],
        axis=1).astype(jnp.bfloat16)
    W_node = jnp.concatenate([W_emb[:_H], W_att[:_H]],
                             axis=1).astype(jnp.bfloat16)
    b_na = jnp.concatenate([b_emb, b_att]).reshape(1, 2 * _MSG)
    W_zr = jnp.concatenate(
        [jnp.concatenate([Wz, Wr, Wh], axis=1),
         jnp.concatenate([Uz, Ur, jnp.zeros((_H, _H), _F32)], axis=1)],
        axis=0).astype(jnp.bfloat16)
    Uh = Uh.astype(jnp.bfloat16)
    b_zr = jnp.concatenate([bz, br, bh]).reshape(1, 3 * _H)
    b_gate2 = b_gate.reshape(1, _H)
    b_er2 = b_er.reshape(1, _H)
    b_out2 = b_out.reshape(1, _H)

    def full(shape):
        return pl.BlockSpec(shape, lambda b: (0,) * len(shape))

    grid_spec = pl.GridSpec(
        grid=(_B // _G,),
        in_specs=[
            pl.BlockSpec((_G, _N, _H), lambda b: (b, 0, 0)),
            pl.BlockSpec((_G, _N * _N, _E), lambda b: (b, 0, 0)),
            full(W_edge.shape), full(W_node.shape), full(b_na.shape),
            full(W_zr.shape), full(b_zr.shape), full(Uh.shape),
            full(W_gate.shape), full(b_gate2.shape),
            full(W_er.shape), full(b_er2.shape),
            full(W_out.shape), full(b_out2.shape),
        ],
        out_specs=pl.BlockSpec((_G, 1, _H), lambda b: (b, 0, 0)),
    )

    out = pl.pallas_call(
        _mpnn_body,
        grid_spec=grid_spec,
        out_shape=jax.ShapeDtypeStruct((_B, 1, _H), _F32),
        compiler_params=pltpu.CompilerParams(
            dimension_semantics=("arbitrary",)),
    )(nodes, e2, W_edge, W_node, b_na, W_zr, b_zr, Uh,
      W_gate, b_gate2, W_er, b_er2, W_out, b_out2)
    return out.reshape(_B, _H)
